# baseline (speedup 1.0000x reference)
import functools

import jax
import jax.numpy as jnp
from jax import lax
from jax.experimental import pallas as pl
from jax.experimental.pallas import tpu as pltpu
from jax.experimental.pallas import tpu_sc as plsc

NC = 2
NS = 16
NW = NC * NS
L = 16

B = 80
ZC = 80

_SC_PARAMS = pltpu.CompilerParams(use_tc_tiling_on_sc=False)


def _zero_2d(ref, nrows, ncols):
    z = jnp.zeros((L,), jnp.float32)

    def row(r, _):
        def col(c, __):
            ref[r, pl.ds(c * L, L)] = z
            return 0

        return lax.fori_loop(0, ncols // L, col, 0)

    lax.fori_loop(0, nrows, row, 0)


def _fill_2d(ref, nrows, ncols, value):
    v = jnp.full((L,), value, jnp.float32)

    def row(r, _):
        def col(c, __):
            ref[r, pl.ds(c * L, L)] = v
            return 0

        return lax.fori_loop(0, ncols // L, col, 0)

    lax.fori_loop(0, nrows, row, 0)


def _make_sc_deg(n_pad, nb):
    rps = n_pad // NS
    mesh = plsc.VectorSubcoreMesh(core_axis_name="c", subcore_axis_name="s")

    def body(ei_hbm, deg_out, dst_v, ones_v, deg_sh, sem):
        c = lax.axis_index("c")
        s = lax.axis_index("s")
        wid = c * NS + s
        row0 = s * rps

        pltpu.sync_copy(ei_hbm.at[1, wid], dst_v)
        _zero_2d(ones_v, B, L)
        for k in range(rps // ZC):
            pltpu.sync_copy(ones_v.at[pl.ds(0, ZC)],
                            deg_sh.at[pl.ds(row0 + k * ZC, ZC)])
        _fill_2d(ones_v, B, L, 1.0)
        plsc.subcore_barrier()

        def fire(j, _):
            pltpu.async_copy(ones_v, deg_sh.at[dst_v.at[j]], sem, add=True)
            return 0

        lax.fori_loop(0, nb, fire, 0)

        def drain(j, _):
            pltpu.make_async_copy(ones_v, deg_sh.at[dst_v.at[j]], sem).wait()
            return 0

        lax.fori_loop(0, nb, drain, 0)
        plsc.subcore_barrier()
        pltpu.sync_copy(deg_sh.at[pl.ds(row0, rps)],
                        deg_out.at[c, pl.ds(row0, rps)])

    return pl.kernel(
        body,
        out_type=jax.ShapeDtypeStruct((NC, n_pad, L), jnp.float32),
        mesh=mesh,
        scratch_types=[
            pltpu.VMEM((nb, B), jnp.int32),
            pltpu.VMEM((B, L), jnp.float32),
            pltpu.VMEM_SHARED((n_pad, L), jnp.float32),
            pltpu.SemaphoreType.DMA,
        ],
        compiler_params=_SC_PARAMS,
    )


def _make_sc_scatter5(n_pad, d, nb, col_split=False):
    assert nb % 5 == 0 and nb >= 15
    rps = n_pad // NS
    mesh = plsc.VectorSubcoreMesh(core_axis_name="c", subcore_axis_name="s")

    def body(p_hbm, ei_hbm, acc_out,
             src_v, dst_v, r0, r1, r2, r3, r4,
             acc_sh, g0, g1, g2, g3, g4, s0, s1, s2, s3, s4):
        bufs = (r0, r1, r2, r3, r4)
        gsems = (g0, g1, g2, g3, g4)
        ssems = (s0, s1, s2, s3, s4)
        c = lax.axis_index("c")
        s = lax.axis_index("s")
        row0 = s * rps

        if col_split:
            pltpu.sync_copy(ei_hbm.at[0, s], src_v)
            pltpu.sync_copy(ei_hbm.at[1, s], dst_v)
            pv = p_hbm.at[c]
        else:
            wid = c * NS + s
            pltpu.sync_copy(ei_hbm.at[0, wid], src_v)
            pltpu.sync_copy(ei_hbm.at[1, wid], dst_v)
            pv = p_hbm

        _zero_2d(r0, B, d)
        for k in range(rps // ZC):
            pltpu.sync_copy(r0.at[pl.ds(0, ZC)],
                            acc_sh.at[pl.ds(row0 + k * ZC, ZC)])
        plsc.subcore_barrier()

        def gather(j, kb):
            pltpu.async_copy(pv.at[src_v.at[j]], bufs[kb], gsems[kb])

        def wait_gather(j, kb):
            pltpu.make_async_copy(pv.at[src_v.at[j]], bufs[kb],
                                  gsems[kb]).wait()

        def scatter(j, kb):
            pltpu.async_copy(bufs[kb], acc_sh.at[dst_v.at[j]], ssems[kb],
                             add=True)

        def wait_scatter(j, kb):
            pltpu.make_async_copy(bufs[kb], acc_sh.at[dst_v.at[j]],
                                  ssems[kb]).wait()

        gather(0, 0)
        gather(1, 1)

        for kb in range(5):
            if kb >= 3:
                wait_scatter(kb - 3, (kb - 3) % 5)
            gather(kb + 2, (kb + 2) % 5)
            wait_gather(kb, kb)
            scatter(kb, kb)

        def group(m, _):
            j0 = m * 5
            for kb in range(5):
                j = j0 + kb
                wait_scatter(j - 3, (kb + 2) % 5)
                gather(j + 2, (kb + 2) % 5)
                wait_gather(j, kb)
                scatter(j, kb)
            return 0

        lax.fori_loop(1, nb // 5 - 1, group, 0)

        jl = nb - 5
        for kb in range(5):
            j = jl + kb
            wait_scatter(j - 3, (kb + 2) % 5)
            if j + 2 <= nb - 1:
                gather(j + 2, (kb + 2) % 5)
            wait_gather(j, kb)
            scatter(j, kb)
        for j in range(nb - 3, nb):
            wait_scatter(j, j % 5)

        plsc.subcore_barrier()
        pltpu.sync_copy(acc_sh.at[pl.ds(row0, rps)],
                        acc_out.at[c, pl.ds(row0, rps)])

    return pl.kernel(
        body,
        out_type=jax.ShapeDtypeStruct((NC, n_pad, d), jnp.float32),
        mesh=mesh,
        scratch_types=(
            [pltpu.VMEM((nb, B), jnp.int32)] * 2
            + [pltpu.VMEM((B, d), jnp.float32)] * 5
            + [pltpu.VMEM_SHARED((n_pad, d), jnp.float32)]
            + [pltpu.SemaphoreType.DMA] * 10
        ),
        compiler_params=_SC_PARAMS,
    )


def _mm_in_body(x_ref, wl_ref, wr_ref, b_ref, p_ref, q_ref):
    x = x_ref[...]
    p = jnp.dot(x, wl_ref[...], precision=lax.Precision.HIGHEST,
                preferred_element_type=jnp.float32)
    dh = p.shape[1] // 2
    p_ref[0] = p[:, :dh]
    p_ref[1] = p[:, dh:]
    q_ref[...] = jnp.dot(x, wr_ref[...], precision=lax.Precision.HIGHEST,
                         preferred_element_type=jnp.float32) + b_ref[...]


def _mid_body(a_ref, dg_ref, q1_ref, wl_ref, wr_ref, b2_ref, p2_ref, q2_ref):
    acc = jnp.concatenate([a_ref[0], a_ref[1]], axis=1)
    deg = dg_ref[0, :, 0:1] + dg_ref[1, :, 0:1]
    inv = 1.0 / jnp.maximum(deg, 1.0)
    h = jnp.maximum(acc * inv + q1_ref[...], 0.0)
    p2_ref[...] = jnp.dot(h, wl_ref[...], precision=lax.Precision.HIGHEST,
                          preferred_element_type=jnp.float32)
    q2_ref[...] = jnp.dot(h, wr_ref[...], precision=lax.Precision.HIGHEST,
                          preferred_element_type=jnp.float32) + b2_ref[...]


def _out_body(n_classes, a_ref, dg_ref, q2_ref, o_ref):
    acc = a_ref[0] + a_ref[1]
    deg = dg_ref[0, :, 0:1] + dg_ref[1, :, 0:1]
    inv = 1.0 / jnp.maximum(deg, 1.0)
    logits = acc * inv + q2_ref[...]
    col = lax.broadcasted_iota(jnp.int32, logits.shape, 1)
    logits = jnp.where(col < n_classes, logits, -1e30)
    m = jnp.max(logits, axis=1, keepdims=True)
    lse = jnp.log(jnp.sum(jnp.exp(logits - m), axis=1, keepdims=True)) + m
    o_ref[...] = (logits - lse)[:, :n_classes]


def kernel(x, edge_index, W1_l, b1, W1_r, W2_l, b2, W2_r):
    n, d_in = x.shape
    d_hid = W1_l.shape[1]
    d_out = W2_l.shape[1]
    e = edge_index.shape[1]

    n_pad = ((n + NS * B - 1) // (NS * B)) * (NS * B)
    d2 = ((d_out + L - 1) // L) * L
    ew = e // NW
    nb = (ew + B - 1) // B
    ew_pad = nb * B

    assert ew_pad == ew, "edge count must divide evenly into blocks"
    nbt = (e // NS) // B
    eif = edge_index.astype(jnp.int32)
    ei = eif.reshape(2, NW, nb, B)
    ei16 = eif.reshape(2, NS, nbt, B)
    d_half = d_hid // 2

    W2_lp = jnp.pad(W2_l, ((0, 0), (0, d2 - d_out)))
    W2_rp = jnp.pad(W2_r, ((0, 0), (0, d2 - d_out)))
    b2p = jnp.pad(b2, (0, d2 - d_out)).reshape(1, d2)

    rows = 2000
    grid = (n // rows,)

    deg = _make_sc_deg(n_pad, nb)(ei)

    p1h, q1 = pl.pallas_call(
        _mm_in_body,
        grid=grid,
        in_specs=[
            pl.BlockSpec((rows, d_in), lambda i: (i, 0)),
            pl.BlockSpec((d_in, d_hid), lambda i: (0, 0)),
            pl.BlockSpec((d_in, d_hid), lambda i: (0, 0)),
            pl.BlockSpec((1, d_hid), lambda i: (0, 0)),
        ],
        out_specs=[
            pl.BlockSpec((NC, rows, d_half), lambda i: (0, i, 0)),
            pl.BlockSpec((rows, d_hid), lambda i: (i, 0)),
        ],
        out_shape=[
            jax.ShapeDtypeStruct((NC, n, d_half), jnp.float32),
            jax.ShapeDtypeStruct((n, d_hid), jnp.float32),
        ],
    )(x, W1_l, W1_r, b1.reshape(1, d_hid))

    acc1 = _make_sc_scatter5(n_pad, d_half, nbt, col_split=True)(p1h, ei16)

    p2, q2 = pl.pallas_call(
        _mid_body,
        grid=grid,
        in_specs=[
            pl.BlockSpec((NC, rows, d_half), lambda i: (0, i, 0)),
            pl.BlockSpec((NC, rows, L), lambda i: (0, i, 0)),
            pl.BlockSpec((rows, d_hid), lambda i: (i, 0)),
            pl.BlockSpec((d_hid, d2), lambda i: (0, 0)),
            pl.BlockSpec((d_hid, d2), lambda i: (0, 0)),
            pl.BlockSpec((1, d2), lambda i: (0, 0)),
        ],
        out_specs=[
            pl.BlockSpec((rows, d2), lambda i: (i, 0)),
            pl.BlockSpec((rows, d2), lambda i: (i, 0)),
        ],
        out_shape=[
            jax.ShapeDtypeStruct((n, d2), jnp.float32),
            jax.ShapeDtypeStruct((n, d2), jnp.float32),
        ],
    )(acc1, deg, q1, W2_lp, W2_rp, b2p)

    acc2 = _make_sc_scatter5(n_pad, d2, nb)(p2, ei)

    out = pl.pallas_call(
        functools.partial(_out_body, d_out),
        grid=grid,
        in_specs=[
            pl.BlockSpec((NC, rows, d2), lambda i: (0, i, 0)),
            pl.BlockSpec((NC, rows, L), lambda i: (0, i, 0)),
            pl.BlockSpec((rows, d2), lambda i: (i, 0)),
        ],
        out_specs=pl.BlockSpec((rows, d_out), lambda i: (i, 0)),
        out_shape=jax.ShapeDtypeStruct((n, d_out), jnp.float32),
    )(acc2, deg, q2)

    return out


# speedup vs baseline: 1.0153x; 1.0153x over previous
import functools

import jax
import jax.numpy as jnp
from jax import lax
from jax.experimental import pallas as pl
from jax.experimental.pallas import tpu as pltpu
from jax.experimental.pallas import tpu_sc as plsc

NC = 2
NS = 16
NW = NC * NS
L = 16

B = 80
ZC = 80

_SC_PARAMS = pltpu.CompilerParams(use_tc_tiling_on_sc=False)


def _zero_2d(ref, nrows, ncols):
    z = jnp.zeros((L,), jnp.float32)

    def row(r, _):
        def col(c, __):
            ref[r, pl.ds(c * L, L)] = z
            return 0

        return lax.fori_loop(0, ncols // L, col, 0)

    lax.fori_loop(0, nrows, row, 0)


def _fill_2d(ref, nrows, ncols, value):
    v = jnp.full((L,), value, jnp.float32)

    def row(r, _):
        def col(c, __):
            ref[r, pl.ds(c * L, L)] = v
            return 0

        return lax.fori_loop(0, ncols // L, col, 0)

    lax.fori_loop(0, nrows, row, 0)


def _make_sc_deg(n_pad, nb):
    rps = n_pad // NS
    mesh = plsc.VectorSubcoreMesh(core_axis_name="c", subcore_axis_name="s")

    def body(ei_hbm, deg_out, dst_v, ones_v, deg_sh, sem):
        c = lax.axis_index("c")
        s = lax.axis_index("s")
        wid = c * NS + s
        row0 = s * rps

        pltpu.sync_copy(ei_hbm.at[1, wid], dst_v)
        _zero_2d(ones_v, B, L)
        for k in range(rps // ZC):
            pltpu.sync_copy(ones_v.at[pl.ds(0, ZC)],
                            deg_sh.at[pl.ds(row0 + k * ZC, ZC)])
        _fill_2d(ones_v, B, L, 1.0)
        plsc.subcore_barrier()

        def fire(j, _):
            pltpu.async_copy(ones_v, deg_sh.at[dst_v.at[j]], sem, add=True)
            return 0

        lax.fori_loop(0, nb, fire, 0)

        def drain(j, _):
            pltpu.make_async_copy(ones_v, deg_sh.at[dst_v.at[j]], sem).wait()
            return 0

        lax.fori_loop(0, nb, drain, 0)
        plsc.subcore_barrier()
        pltpu.sync_copy(deg_sh.at[pl.ds(row0, rps)],
                        deg_out.at[c, pl.ds(row0, rps)])

    return pl.kernel(
        body,
        out_type=jax.ShapeDtypeStruct((NC, n_pad, L), jnp.float32),
        mesh=mesh,
        scratch_types=[
            pltpu.VMEM((nb, B), jnp.int32),
            pltpu.VMEM((B, L), jnp.float32),
            pltpu.VMEM_SHARED((n_pad, L), jnp.float32),
            pltpu.SemaphoreType.DMA,
        ],
        compiler_params=_SC_PARAMS,
    )


def _make_sc_scatter5(n_pad, d, nb, col_split=False):
    assert nb % 5 == 0 and nb >= 15
    rps = n_pad // NS
    mesh = plsc.VectorSubcoreMesh(core_axis_name="c", subcore_axis_name="s")

    def body(p_hbm, ei_hbm, acc_out,
             src_v, dst_v, r0, r1, r2, r3, r4,
             acc_sh, g0, g1, g2, g3, g4, s0, s1, s2, s3, s4):
        bufs = (r0, r1, r2, r3, r4)
        gsems = (g0, g1, g2, g3, g4)
        ssems = (s0, s1, s2, s3, s4)
        c = lax.axis_index("c")
        s = lax.axis_index("s")
        row0 = s * rps

        if col_split:
            pltpu.sync_copy(ei_hbm.at[0, s], src_v)
            pltpu.sync_copy(ei_hbm.at[1, s], dst_v)
            nbv, bv = src_v.shape

            def fix_row(r, _):
                def fix_col(k, __):
                    sl = src_v[r, pl.ds(k * L, L)]
                    src_v[r, pl.ds(k * L, L)] = sl * 2 + c
                    return 0

                return lax.fori_loop(0, bv // L, fix_col, 0)

            lax.fori_loop(0, nbv, fix_row, 0)
            pv = p_hbm
        else:
            wid = c * NS + s
            pltpu.sync_copy(ei_hbm.at[0, wid], src_v)
            pltpu.sync_copy(ei_hbm.at[1, wid], dst_v)
            pv = p_hbm

        _zero_2d(r0, B, d)
        for k in range(rps // ZC):
            pltpu.sync_copy(r0.at[pl.ds(0, ZC)],
                            acc_sh.at[pl.ds(row0 + k * ZC, ZC)])
        plsc.subcore_barrier()

        def gather(j, kb):
            pltpu.async_copy(pv.at[src_v.at[j]], bufs[kb], gsems[kb])

        def wait_gather(j, kb):
            pltpu.make_async_copy(pv.at[src_v.at[j]], bufs[kb],
                                  gsems[kb]).wait()

        def scatter(j, kb):
            pltpu.async_copy(bufs[kb], acc_sh.at[dst_v.at[j]], ssems[kb],
                             add=True)

        def wait_scatter(j, kb):
            pltpu.make_async_copy(bufs[kb], acc_sh.at[dst_v.at[j]],
                                  ssems[kb]).wait()

        gather(0, 0)
        gather(1, 1)

        for kb in range(5):
            if kb >= 3:
                wait_scatter(kb - 3, (kb - 3) % 5)
            gather(kb + 2, (kb + 2) % 5)
            wait_gather(kb, kb)
            scatter(kb, kb)

        def group(m, _):
            j0 = m * 5
            for kb in range(5):
                j = j0 + kb
                wait_scatter(j - 3, (kb + 2) % 5)
                gather(j + 2, (kb + 2) % 5)
                wait_gather(j, kb)
                scatter(j, kb)
            return 0

        lax.fori_loop(1, nb // 5 - 1, group, 0)

        jl = nb - 5
        for kb in range(5):
            j = jl + kb
            wait_scatter(j - 3, (kb + 2) % 5)
            if j + 2 <= nb - 1:
                gather(j + 2, (kb + 2) % 5)
            wait_gather(j, kb)
            scatter(j, kb)
        for j in range(nb - 3, nb):
            wait_scatter(j, j % 5)

        plsc.subcore_barrier()
        pltpu.sync_copy(acc_sh.at[pl.ds(row0, rps)],
                        acc_out.at[c, pl.ds(row0, rps)])

    return pl.kernel(
        body,
        out_type=jax.ShapeDtypeStruct((NC, n_pad, d), jnp.float32),
        mesh=mesh,
        scratch_types=(
            [pltpu.VMEM((nb, B), jnp.int32)] * 2
            + [pltpu.VMEM((B, d), jnp.float32)] * 5
            + [pltpu.VMEM_SHARED((n_pad, d), jnp.float32)]
            + [pltpu.SemaphoreType.DMA] * 10
        ),
        compiler_params=_SC_PARAMS,
    )


def _mm_in_body(x_ref, wl_ref, wr_ref, b_ref, p_ref, q_ref):
    x = x_ref[...]
    p_ref[...] = jnp.dot(x, wl_ref[...], precision=lax.Precision.HIGHEST,
                         preferred_element_type=jnp.float32)
    q_ref[...] = jnp.dot(x, wr_ref[...], precision=lax.Precision.HIGHEST,
                         preferred_element_type=jnp.float32) + b_ref[...]


def _mid_body(a_ref, dg_ref, q1_ref, wl_ref, wr_ref, b2_ref, p2_ref, q2_ref):
    acc = jnp.concatenate([a_ref[0], a_ref[1]], axis=1)
    deg = dg_ref[0, :, 0:1] + dg_ref[1, :, 0:1]
    inv = 1.0 / jnp.maximum(deg, 1.0)
    h = jnp.maximum(acc * inv + q1_ref[...], 0.0)
    p2_ref[...] = jnp.dot(h, wl_ref[...], precision=lax.Precision.HIGHEST,
                          preferred_element_type=jnp.float32)
    q2_ref[...] = jnp.dot(h, wr_ref[...], precision=lax.Precision.HIGHEST,
                          preferred_element_type=jnp.float32) + b2_ref[...]


def _out_body(n_classes, a_ref, dg_ref, q2_ref, o_ref):
    acc = a_ref[0] + a_ref[1]
    deg = dg_ref[0, :, 0:1] + dg_ref[1, :, 0:1]
    inv = 1.0 / jnp.maximum(deg, 1.0)
    logits = acc * inv + q2_ref[...]
    col = lax.broadcasted_iota(jnp.int32, logits.shape, 1)
    logits = jnp.where(col < n_classes, logits, -1e30)
    m = jnp.max(logits, axis=1, keepdims=True)
    lse = jnp.log(jnp.sum(jnp.exp(logits - m), axis=1, keepdims=True)) + m
    o_ref[...] = (logits - lse)[:, :n_classes]


def kernel(x, edge_index, W1_l, b1, W1_r, W2_l, b2, W2_r):
    n, d_in = x.shape
    d_hid = W1_l.shape[1]
    d_out = W2_l.shape[1]
    e = edge_index.shape[1]

    n_pad = ((n + NS * B - 1) // (NS * B)) * (NS * B)
    d2 = ((d_out + L - 1) // L) * L
    ew = e // NW
    nb = (ew + B - 1) // B
    ew_pad = nb * B

    assert ew_pad == ew, "edge count must divide evenly into blocks"
    nbt = (e // NS) // B
    eif = edge_index.astype(jnp.int32)
    ei = eif.reshape(2, NW, nb, B)
    ei16 = eif.reshape(2, NS, nbt, B)
    d_half = d_hid // 2

    W2_lp = jnp.pad(W2_l, ((0, 0), (0, d2 - d_out)))
    W2_rp = jnp.pad(W2_r, ((0, 0), (0, d2 - d_out)))
    b2p = jnp.pad(b2, (0, d2 - d_out)).reshape(1, d2)

    rows = 2000
    grid = (n // rows,)

    deg = _make_sc_deg(n_pad, nb)(ei)

    p1, q1 = pl.pallas_call(
        _mm_in_body,
        grid=grid,
        in_specs=[
            pl.BlockSpec((rows, d_in), lambda i: (i, 0)),
            pl.BlockSpec((d_in, d_hid), lambda i: (0, 0)),
            pl.BlockSpec((d_in, d_hid), lambda i: (0, 0)),
            pl.BlockSpec((1, d_hid), lambda i: (0, 0)),
        ],
        out_specs=[
            pl.BlockSpec((rows, d_hid), lambda i: (i, 0)),
            pl.BlockSpec((rows, d_hid), lambda i: (i, 0)),
        ],
        out_shape=[
            jax.ShapeDtypeStruct((n, d_hid), jnp.float32),
            jax.ShapeDtypeStruct((n, d_hid), jnp.float32),
        ],
    )(x, W1_l, W1_r, b1.reshape(1, d_hid))

    acc1 = _make_sc_scatter5(n_pad, d_half, nbt, col_split=True)(
        p1.reshape(2 * n, d_half), ei16)

    p2, q2 = pl.pallas_call(
        _mid_body,
        grid=grid,
        in_specs=[
            pl.BlockSpec((NC, rows, d_half), lambda i: (0, i, 0)),
            pl.BlockSpec((NC, rows, L), lambda i: (0, i, 0)),
            pl.BlockSpec((rows, d_hid), lambda i: (i, 0)),
            pl.BlockSpec((d_hid, d2), lambda i: (0, 0)),
            pl.BlockSpec((d_hid, d2), lambda i: (0, 0)),
            pl.BlockSpec((1, d2), lambda i: (0, 0)),
        ],
        out_specs=[
            pl.BlockSpec((rows, d2), lambda i: (i, 0)),
            pl.BlockSpec((rows, d2), lambda i: (i, 0)),
        ],
        out_shape=[
            jax.ShapeDtypeStruct((n, d2), jnp.float32),
            jax.ShapeDtypeStruct((n, d2), jnp.float32),
        ],
    )(acc1, deg, q1, W2_lp, W2_rp, b2p)

    acc2 = _make_sc_scatter5(n_pad, d2, nb)(p2, ei)

    out = pl.pallas_call(
        functools.partial(_out_body, d_out),
        grid=grid,
        in_specs=[
            pl.BlockSpec((NC, rows, d2), lambda i: (0, i, 0)),
            pl.BlockSpec((NC, rows, L), lambda i: (0, i, 0)),
            pl.BlockSpec((rows, d2), lambda i: (i, 0)),
        ],
        out_specs=pl.BlockSpec((rows, d_out), lambda i: (i, 0)),
        out_shape=jax.ShapeDtypeStruct((n, d_out), jnp.float32),
    )(acc2, deg, q2)

    return out
